# baseline (speedup 1.0000x reference)
import jax
import jax.numpy as jnp
import numpy as np
from jax import lax
from jax.experimental import pallas as pl
from jax.experimental.pallas import tpu as pltpu
from jax.experimental.pallas import tpu_sc as plsc

EMBED_DIM = 32
SCALE = float(np.sqrt(np.float32(EMBED_DIM)))

NROWS = 16384
SEQ = 50
NUM_WORKERS = 32
B_PER_W = NROWS // NUM_WORKERS
JBLK = 128
J_PER_W = B_PER_W // JBLK
N_CHUNKS = SEQ * J_PER_W
NBUF = 4

VOCAB = 1000000
PTOK = 128
P_CHUNKS = 7812
PTAIL = VOCAB - P_CHUNKS * PTOK
PBUF = 4


def _prep_body(tt_hbm, tail_hbm, lin_hbm, g_v, t_v, *sems):
    isems = sems[:PBUF]
    osems = sems[PBUF:]
    wid = lax.axis_index("s") * 2 + lax.axis_index("c")
    n_j = (P_CHUNKS - wid + NUM_WORKERS - 1) // NUM_WORKERS

    def fire_in(t, b):
        j = wid + t * NUM_WORKERS
        pltpu.async_copy(tt_hbm.at[:, pl.ds(j * PTOK, PTOK)],
                         g_v.at[b, :, pl.ds(0, PTOK)], isems[b])

    def wait_in(t, b):
        j = wid + t * NUM_WORKERS
        pltpu.make_async_copy(tt_hbm.at[:, pl.ds(j * PTOK, PTOK)],
                              g_v.at[b, :, pl.ds(0, PTOK)], isems[b]).wait()

    def fire_out(t, b):
        j = wid + t * NUM_WORKERS
        pltpu.async_copy(
            t_v.at[b], lin_hbm.at[pl.ds(j * (PTOK // 4), PTOK // 4)],
            osems[b])

    def wait_out(t, b):
        j = wid + t * NUM_WORKERS
        pltpu.make_async_copy(
            t_v.at[b], lin_hbm.at[pl.ds(j * (PTOK // 4), PTOK // 4)],
            osems[b]).wait()

    def transpose_chunk(b):
        iota = lax.iota(jnp.int32, 16)
        didx = [iota, iota + 16]

        def tok(l, carry):
            cl = jnp.full((16,), l, jnp.int32)
            row = l // 4
            col = (l % 4) * EMBED_DIM
            for h in range(2):
                vals = plsc.load_gather(g_v.at[b], [didx[h], cl])
                t_v[b, row, pl.ds(col + 16 * h, 16)] = vals
            return carry

        lax.fori_loop(0, PTOK, tok, 0, unroll=8)

    @pl.when(wid == 0)
    def _():
        pltpu.sync_copy(tail_hbm, t_v.at[0, pl.ds(0, PTAIL // 4)])
        pltpu.sync_copy(t_v.at[0, pl.ds(0, PTAIL // 4)],
                        lin_hbm.at[pl.ds(P_CHUNKS * (PTOK // 4), PTAIL // 4)])

    for b in range(PBUF):
        @pl.when(b < n_j)
        def _():
            fire_in(b, b)

    @pl.loop(0, P_CHUNKS // NUM_WORKERS + 1, step=PBUF)
    def step(t0):
        for b in range(PBUF):
            t = t0 + b

            @pl.when(t < n_j)
            def _():
                wait_in(t, b)

                @pl.when(t >= PBUF)
                def _():
                    wait_out(t - PBUF, b)

                transpose_chunk(b)
                fire_out(t, b)

                @pl.when(t + PBUF < n_j)
                def _():
                    fire_in(t + PBUF, b)

    for b in range(PBUF):
        last_t = ((n_j - 1 - b) // PBUF) * PBUF + b
        wait_out(last_t, b)


def _embed_body(xt_hbm, table_hbm, out_hbm, idx_v, g_v, t_v, *sems):
    gsems = sems[:NBUF]
    osems = sems[NBUF:]
    wid = lax.axis_index("s") * 2 + lax.axis_index("c")
    b0 = wid * B_PER_W

    for k in range(J_PER_W):
        pltpu.sync_copy(xt_hbm.at[:, pl.ds(b0 + k * JBLK, JBLK)],
                        idx_v.at[pl.ds(k * SEQ, SEQ)])

    def fire_gather(c, b):
        pltpu.async_copy(table_hbm.at[idx_v.at[c]], g_v.at[b], gsems[b])

    def wait_gather(c, b):
        pltpu.make_async_copy(
            table_hbm.at[idx_v.at[c]], g_v.at[b], gsems[b]).wait()

    def out_slices(c):
        k = c // SEQ
        s = c - k * SEQ
        return s, (wid * J_PER_W + k)

    def fire_out(c, b):
        s, j = out_slices(c)
        pltpu.async_copy(
            t_v.at[b, :, :, pl.ds(0, JBLK)], out_hbm.at[s, :, j], osems[b])

    def wait_out(c, b):
        s, j = out_slices(c)
        pltpu.make_async_copy(
            t_v.at[b, :, :, pl.ds(0, JBLK)], out_hbm.at[s, :, j],
            osems[b]).wait()

    def transpose_scale(b):
        iota = lax.iota(jnp.int32, 16)
        didx = [iota, iota + 16]
        ci = [d >> 3 for d in didx]
        cr = [d & 7 for d in didx]

        def tok(l, carry):
            cl = jnp.full((16,), l, jnp.int32)
            for h in range(2):
                vals = g_v[b, l, pl.ds(16 * h, 16)]
                plsc.store_scatter(t_v.at[b], [ci[h], cr[h], cl],
                                   vals * SCALE)
            return carry

        lax.fori_loop(0, JBLK, tok, 0, unroll=8)

    for b in range(NBUF):
        fire_gather(b, b)

    @pl.loop(0, N_CHUNKS, step=NBUF)
    def step(c0):
        for b in range(NBUF):
            c = c0 + b
            wait_gather(c, b)

            @pl.when(c >= NBUF)
            def _():
                wait_out(c - NBUF, b)

            transpose_scale(b)
            fire_out(c, b)

            @pl.when(c + NBUF < N_CHUNKS)
            def _():
                fire_gather(c + NBUF, b)

    for b in range(NBUF):
        wait_out(N_CHUNKS - NBUF + b, b)


@jax.jit
def _prep(tt, tail):
    mesh = plsc.VectorSubcoreMesh(core_axis_name="c", subcore_axis_name="s")
    f = pl.kernel(
        _prep_body,
        mesh=mesh,
        out_type=jax.ShapeDtypeStruct((VOCAB // 4, 128), jnp.float32),
        scratch_types=[
            pltpu.VMEM((PBUF, EMBED_DIM, PTOK + 1), jnp.float32),
            pltpu.VMEM((PBUF, PTOK // 4, PTOK), jnp.float32),
        ] + [pltpu.SemaphoreType.DMA] * (2 * PBUF),
        compiler_params=pltpu.CompilerParams(
            use_tc_tiling_on_sc=True, needs_layout_passes=False),
    )
    return f(tt, tail)


@jax.jit
def _embed(xt, table):
    mesh = plsc.VectorSubcoreMesh(core_axis_name="c", subcore_axis_name="s")
    f = pl.kernel(
        _embed_body,
        mesh=mesh,
        out_type=jax.ShapeDtypeStruct((SEQ, 4, NROWS // JBLK, 8, JBLK),
                                      jnp.float32),
        scratch_types=[
            pltpu.VMEM((N_CHUNKS, JBLK), jnp.int32),
            pltpu.VMEM((NBUF, JBLK, EMBED_DIM), jnp.float32),
            pltpu.VMEM((NBUF, 4, 8, JBLK + 1), jnp.float32),
        ] + [pltpu.SemaphoreType.DMA] * (2 * NBUF),
        compiler_params=pltpu.CompilerParams(
            use_tc_tiling_on_sc=False, needs_layout_passes=False),
    )
    return f(xt, table)


def kernel(x, table):
    xt = x.T.astype(jnp.int32)
    tail = table[P_CHUNKS * PTOK:, :].reshape(PTAIL // 4, 128)
    lin4 = _prep(table.T, tail)
    table_lin = lin4.reshape(VOCAB, EMBED_DIM)
    x5 = _embed(xt, table_lin)
    out = jnp.transpose(x5, (2, 4, 0, 1, 3)).reshape(NROWS, SEQ, EMBED_DIM)
    return out


# speedup vs baseline: 1.3462x; 1.3462x over previous
import jax
import jax.numpy as jnp
import numpy as np
from jax import lax
from jax.experimental import pallas as pl
from jax.experimental.pallas import tpu as pltpu
from jax.experimental.pallas import tpu_sc as plsc

EMBED_DIM = 32
SCALE = float(np.sqrt(np.float32(EMBED_DIM)))

NROWS = 16384
SEQ = 50
NUM_WORKERS = 32
B_PER_W = NROWS // NUM_WORKERS
JBLK = 128
J_PER_W = B_PER_W // JBLK
N_CHUNKS = SEQ * J_PER_W
NBUF = 8


def _embed_body(xt_hbm, table_hbm, out_hbm, idx_v, g_v, t_v, *sems):
    gsems = sems[:NBUF]
    osems = sems[NBUF:]
    wid = lax.axis_index("s") * 2 + lax.axis_index("c")
    b0 = wid * B_PER_W

    for k in range(J_PER_W):
        pltpu.sync_copy(xt_hbm.at[:, pl.ds(b0 + k * JBLK, JBLK)],
                        idx_v.at[pl.ds(k * SEQ, SEQ)])

    def fire_gather(c, b):
        pltpu.async_copy(table_hbm.at[idx_v.at[c]], g_v.at[b], gsems[b])

    def wait_gather(c, b):
        pltpu.make_async_copy(
            table_hbm.at[idx_v.at[c]], g_v.at[b], gsems[b]).wait()

    def out_slices(c):
        k = c // SEQ
        s = c - k * SEQ
        return s, (wid * J_PER_W + k)

    def fire_out(c, b):
        s, j = out_slices(c)
        pltpu.async_copy(
            t_v.at[b, :, :, pl.ds(0, JBLK)], out_hbm.at[s, :, j], osems[b])

    def wait_out(c, b):
        s, j = out_slices(c)
        pltpu.make_async_copy(
            t_v.at[b, :, :, pl.ds(0, JBLK)], out_hbm.at[s, :, j],
            osems[b]).wait()

    def transpose_scale(b):
        iota = lax.iota(jnp.int32, 16)
        didx = [iota, iota + 16]
        ci = [d >> 3 for d in didx]
        cr = [d & 7 for d in didx]

        def tok(l, carry):
            cl = jnp.full((16,), l, jnp.int32)
            for h in range(2):
                vals = g_v[b, l, pl.ds(16 * h, 16)]
                plsc.store_scatter(t_v.at[b], [ci[h], cr[h], cl],
                                   vals * SCALE)
            return carry

        lax.fori_loop(0, JBLK, tok, 0, unroll=8)

    for b in range(NBUF):
        fire_gather(b, b)

    @pl.loop(0, N_CHUNKS, step=NBUF)
    def step(c0):
        for b in range(NBUF):
            c = c0 + b
            wait_gather(c, b)

            @pl.when(c >= NBUF)
            def _():
                wait_out(c - NBUF, b)

            transpose_scale(b)
            fire_out(c, b)

            @pl.when(c + NBUF < N_CHUNKS)
            def _():
                fire_gather(c + NBUF, b)

    for b in range(NBUF):
        wait_out(N_CHUNKS - NBUF + b, b)


@jax.jit
def _embed(xt, table):
    mesh = plsc.VectorSubcoreMesh(core_axis_name="c", subcore_axis_name="s")
    f = pl.kernel(
        _embed_body,
        mesh=mesh,
        out_type=jax.ShapeDtypeStruct((SEQ, 4, NROWS // JBLK, 8, JBLK),
                                      jnp.float32),
        scratch_types=[
            pltpu.VMEM((N_CHUNKS, JBLK), jnp.int32),
            pltpu.VMEM((NBUF, JBLK, EMBED_DIM), jnp.float32),
            pltpu.VMEM((NBUF, 4, 8, JBLK + 1), jnp.float32),
        ] + [pltpu.SemaphoreType.DMA] * (2 * NBUF),
        compiler_params=pltpu.CompilerParams(
            use_tc_tiling_on_sc=False, needs_layout_passes=False),
    )
    return f(xt, table)


def kernel(x, table):
    xt = x.T.astype(jnp.int32)
    x5 = _embed(xt, table)
    out = jnp.transpose(x5, (2, 4, 0, 1, 3)).reshape(NROWS, SEQ, EMBED_DIM)
    return out


# speedup vs baseline: 1.3646x; 1.0137x over previous
import jax
import jax.numpy as jnp
import numpy as np
from jax import lax
from jax.experimental import pallas as pl
from jax.experimental.pallas import tpu as pltpu
from jax.experimental.pallas import tpu_sc as plsc

EMBED_DIM = 32
SCALE = float(np.sqrt(np.float32(EMBED_DIM)))

NROWS = 16384
SEQ = 50
NUM_WORKERS = 32
B_PER_W = NROWS // NUM_WORKERS
JBLK = 128
J_PER_W = B_PER_W // JBLK
N_CHUNKS = SEQ * J_PER_W
NBUF = 4


def _embed_body(xt_hbm, table_hbm, out_hbm, idx_v, g_v, t_v, *sems):
    gsems = sems[:NBUF]
    osems = sems[NBUF:]
    wid = lax.axis_index("s") * 2 + lax.axis_index("c")
    b0 = wid * B_PER_W

    for k in range(J_PER_W):
        pltpu.sync_copy(xt_hbm.at[:, pl.ds(b0 + k * JBLK, JBLK)],
                        idx_v.at[pl.ds(k * SEQ, SEQ)])

    def fire_gather(c, b):
        pltpu.async_copy(table_hbm.at[idx_v.at[c]], g_v.at[b], gsems[b])

    def wait_gather(c, b):
        pltpu.make_async_copy(
            table_hbm.at[idx_v.at[c]], g_v.at[b], gsems[b]).wait()

    def out_slices(c):
        k = c // SEQ
        s = c - k * SEQ
        return s, (wid * J_PER_W + k)

    def fire_out(c, b):
        s, j = out_slices(c)
        pltpu.async_copy(
            t_v.at[b, :, :, pl.ds(0, JBLK)], out_hbm.at[s, :, j], osems[b])

    def wait_out(c, b):
        s, j = out_slices(c)
        pltpu.make_async_copy(
            t_v.at[b, :, :, pl.ds(0, JBLK)], out_hbm.at[s, :, j],
            osems[b]).wait()

    def transpose_scale(b):
        iota = lax.iota(jnp.int32, 16)
        didx = [iota, iota + 16]
        ci = [d >> 3 for d in didx]
        cr = [d & 7 for d in didx]

        def tok(l, carry):
            cl = jnp.full((16,), l, jnp.int32)
            for h in range(2):
                vals = g_v[b, l, pl.ds(16 * h, 16)]
                plsc.store_scatter(t_v.at[b], [ci[h], cr[h], cl],
                                   vals * SCALE)
            return carry

        lax.fori_loop(0, JBLK, tok, 0, unroll=8)

    for b in range(NBUF):
        fire_gather(b, b)

    @pl.loop(0, N_CHUNKS, step=NBUF)
    def step(c0):
        for b in range(NBUF):
            c = c0 + b
            wait_gather(c, b)

            @pl.when(c >= NBUF)
            def _():
                wait_out(c - NBUF, b)

            transpose_scale(b)
            fire_out(c, b)

            @pl.when(c + NBUF < N_CHUNKS)
            def _():
                fire_gather(c + NBUF, b)

    for b in range(NBUF):
        wait_out(N_CHUNKS - NBUF + b, b)


@jax.jit
def _embed(xt, table):
    mesh = plsc.VectorSubcoreMesh(core_axis_name="c", subcore_axis_name="s")
    f = pl.kernel(
        _embed_body,
        mesh=mesh,
        out_type=jax.ShapeDtypeStruct((SEQ, 4, NROWS // JBLK, 8, JBLK),
                                      jnp.float32),
        scratch_types=[
            pltpu.VMEM((N_CHUNKS, JBLK), jnp.int32),
            pltpu.VMEM((NBUF, JBLK, EMBED_DIM), jnp.float32),
            pltpu.VMEM((NBUF, 4, 8, JBLK + 1), jnp.float32),
        ] + [pltpu.SemaphoreType.DMA] * (2 * NBUF),
        compiler_params=pltpu.CompilerParams(
            use_tc_tiling_on_sc=False, needs_layout_passes=False),
    )
    return f(xt, table)


def kernel(x, table):
    xt = x.T.astype(jnp.int32)
    x5 = _embed(xt, table)
    out = jnp.transpose(x5, (2, 4, 0, 1, 3)).reshape(NROWS, SEQ, EMBED_DIM)
    return out
